# initial kernel scaffold (unmeasured)
import jax
import jax.numpy as jnp
from jax import lax
from jax.experimental import pallas as pl
from jax.experimental.pallas import tpu as pltpu

N_DEV = 4
S = 1024
D = 1024
HQ = 8
DH = 128
BLK = 64
SCALE = 0.08838834764831843


def kernel(x, Wq, K_ext, V_ext, Wo):
    x2 = x[0].astype(jnp.bfloat16)
    k2 = K_ext[0].reshape(S, D).astype(jnp.bfloat16)
    v2 = V_ext[0].reshape(S, D).astype(jnp.bfloat16)
    wq = Wq.astype(jnp.bfloat16)
    wo = Wo.astype(jnp.bfloat16)

    def body(x_ref, wq_ref, k_ref, v_ref, wo_ref, out_ref,
             kvf_ref, q_ref, ctx_ref, send_sems, recv_sems):
        my = lax.axis_index("i")
        left = lax.rem(my + (N_DEV - 1), N_DEV)
        right = lax.rem(my + 1, N_DEV)

        barrier = pltpu.get_barrier_semaphore()
        for nbr in (left, right):
            pl.semaphore_signal(barrier, inc=1, device_id=(nbr,),
                                device_id_type=pl.DeviceIdType.MESH)
        pl.semaphore_wait(barrier, 2)

        kvf_ref[0:S, 0:D] = k_ref[...]
        kvf_ref[0:S, D:2 * D] = v_ref[...]

        q_ref[...] = jnp.dot(
            x_ref[...], wq_ref[...], preferred_element_type=jnp.float32
        ).astype(jnp.bfloat16)

        for h in range(N_DEV - 1):
            rdma = pltpu.make_async_remote_copy(
                src_ref=kvf_ref.at[pl.ds(h * S, S), :],
                dst_ref=kvf_ref.at[pl.ds((h + 1) * S, S), :],
                send_sem=send_sems.at[h],
                recv_sem=recv_sems.at[h],
                device_id=(right,),
                device_id_type=pl.DeviceIdType.MESH,
            )
            rdma.start()
            rdma.wait()

        origins = [lax.rem(my - d + N_DEV, N_DEV) for d in range(N_DEV)]

        for hd in range(HQ):
            qh = q_ref[:, hd * DH:(hd + 1) * DH]
            parts = []
            for d in range(N_DEV):
                kh = kvf_ref[d * S:(d + 1) * S, hd * DH:(hd + 1) * DH]
                sd = lax.dot_general(
                    qh, kh, (((1,), (1,)), ((), ())),
                    preferred_element_type=jnp.float32)
                rows = lax.broadcasted_iota(jnp.int32, (S, S), 0)
                cols = lax.broadcasted_iota(jnp.int32, (S, S), 1)
                qb = (rows + my * S) // BLK
                kb = (cols + origins[d] * S) // BLK
                mask = (qb == kb) | (kb == 0) | (lax.rem(qb + kb, 3) == 0)
                parts.append(jnp.where(mask, sd * SCALE, -1e9))
            s = jnp.concatenate(parts, axis=1)
            m = jnp.max(s, axis=1, keepdims=True)
            w = jnp.exp(s - m)
            denom = jnp.sum(w, axis=1, keepdims=True)
            p = (w / denom).astype(jnp.bfloat16)
            vh = kvf_ref[:, D + hd * DH:D + (hd + 1) * DH]
            ctx_ref[:, hd * DH:(hd + 1) * DH] = jnp.dot(
                p, vh, preferred_element_type=jnp.float32).astype(jnp.bfloat16)

        out_ref[...] = jnp.dot(
            ctx_ref[...], wo_ref[...], preferred_element_type=jnp.float32)

    out = pl.pallas_call(
        body,
        out_shape=jax.ShapeDtypeStruct((S, D), jnp.float32),
        in_specs=[pl.BlockSpec(memory_space=pltpu.VMEM)] * 5,
        out_specs=pl.BlockSpec(memory_space=pltpu.VMEM),
        scratch_shapes=[
            pltpu.VMEM((N_DEV * S, 2 * D), jnp.bfloat16),
            pltpu.VMEM((S, D), jnp.bfloat16),
            pltpu.VMEM((S, D), jnp.bfloat16),
            pltpu.SemaphoreType.DMA((N_DEV - 1,)),
            pltpu.SemaphoreType.DMA((N_DEV - 1,)),
        ],
        compiler_params=pltpu.CompilerParams(collective_id=0),
    )(x2, wq, k2, v2, wo)
    return out[None]


# baseline (device time: 308173 ns/iter reference)
import jax
import jax.numpy as jnp
from jax import lax
from jax.experimental import pallas as pl
from jax.experimental.pallas import tpu as pltpu

N_DEV = 4
S = 1024
D = 1024
HQ = 8
DH = 128
BLK = 64
SCALE = 0.08838834764831843


def kernel(x, Wq, K_ext, V_ext, Wo):
    x2 = x[0].astype(jnp.bfloat16)
    k2 = K_ext[0].reshape(S, D).astype(jnp.bfloat16)
    v2 = V_ext[0].reshape(S, D).astype(jnp.bfloat16)
    wq = Wq.astype(jnp.bfloat16)
    wo = Wo.astype(jnp.bfloat16)

    def body(x_ref, wq_ref, k_ref, v_ref, wo_ref, out_ref,
             kvf_ref, q_ref, ctx_ref, send_sems, recv_sems):
        my = lax.axis_index("i")
        left = lax.rem(my + (N_DEV - 1), N_DEV)
        right = lax.rem(my + 1, N_DEV)

        barrier = pltpu.get_barrier_semaphore()
        for nbr in (left, right):
            pl.semaphore_signal(barrier, inc=1, device_id=(nbr,),
                                device_id_type=pl.DeviceIdType.MESH)
        pl.semaphore_wait(barrier, 2)

        kvf_ref[0:S, 0:D] = k_ref[...]
        kvf_ref[0:S, D:2 * D] = v_ref[...]

        q_ref[...] = jnp.dot(
            x_ref[...], wq_ref[...], preferred_element_type=jnp.float32
        ).astype(jnp.bfloat16)

        for h in range(N_DEV - 1):
            rdma = pltpu.make_async_remote_copy(
                src_ref=kvf_ref.at[pl.ds(h * S, S), :],
                dst_ref=kvf_ref.at[pl.ds((h + 1) * S, S), :],
                send_sem=send_sems.at[h],
                recv_sem=recv_sems.at[h],
                device_id=(right,),
                device_id_type=pl.DeviceIdType.MESH,
            )
            rdma.start()
            rdma.wait()

        def head_step(hd, _):
            qh = q_ref[:, pl.ds(hd * DH, DH)]

            def chunk_step(d, carry):
                m_run, l_run, acc = carry
                o = lax.rem(my - d + N_DEV, N_DEV)
                kh = kvf_ref[pl.ds(d * S, S), pl.ds(hd * DH, DH)]
                sd = lax.dot_general(
                    qh, kh, (((1,), (1,)), ((), ())),
                    preferred_element_type=jnp.float32)
                rows = lax.broadcasted_iota(jnp.int32, (S, S), 0)
                cols = lax.broadcasted_iota(jnp.int32, (S, S), 1)
                qb = (rows + my * S) // BLK
                kb = (cols + o * S) // BLK
                mask = (qb == kb) | (kb == 0) | (lax.rem(qb + kb, 3) == 0)
                sd = jnp.where(mask, sd * SCALE, -1e9)
                m_new = jnp.maximum(
                    m_run, jnp.max(sd, axis=1, keepdims=True))
                alpha = jnp.exp(m_run - m_new)
                w = jnp.exp(sd - m_new)
                l_new = l_run * alpha + jnp.sum(w, axis=1, keepdims=True)
                vh = kvf_ref[pl.ds(d * S, S), pl.ds(D + hd * DH, DH)]
                acc_new = acc * alpha + jnp.dot(
                    w.astype(jnp.bfloat16), vh,
                    preferred_element_type=jnp.float32)
                return m_new, l_new, acc_new

            init = (jnp.full((S, 1), -1e30, jnp.float32),
                    jnp.zeros((S, 1), jnp.float32),
                    jnp.zeros((S, DH), jnp.float32))
            m_run, l_run, acc = lax.fori_loop(0, N_DEV, chunk_step, init)
            ctx_ref[:, pl.ds(hd * DH, DH)] = (acc / l_run).astype(jnp.bfloat16)
            return 0

        lax.fori_loop(0, HQ, head_step, 0)

        out_ref[...] = jnp.dot(
            ctx_ref[...], wo_ref[...], preferred_element_type=jnp.float32)

    out = pl.pallas_call(
        body,
        out_shape=jax.ShapeDtypeStruct((S, D), jnp.float32),
        in_specs=[pl.BlockSpec(memory_space=pltpu.VMEM)] * 5,
        out_specs=pl.BlockSpec(memory_space=pltpu.VMEM),
        scratch_shapes=[
            pltpu.VMEM((N_DEV * S, 2 * D), jnp.bfloat16),
            pltpu.VMEM((S, D), jnp.bfloat16),
            pltpu.VMEM((S, D), jnp.bfloat16),
            pltpu.SemaphoreType.DMA((N_DEV - 1,)),
            pltpu.SemaphoreType.DMA((N_DEV - 1,)),
        ],
        compiler_params=pltpu.CompilerParams(
            collective_id=0, vmem_limit_bytes=100 * 1024 * 1024
        ),
    )(x2, wq, k2, v2, wo)
    return out[None]


# device time: 186284 ns/iter; 1.6543x vs baseline; 1.6543x over previous
import jax
import jax.numpy as jnp
from jax import lax
from jax.experimental import pallas as pl
from jax.experimental.pallas import tpu as pltpu

N_DEV = 4
S = 1024
D = 1024
HQ = 8
DH = 128
BLK = 64
SCALE = 0.08838834764831843


def kernel(x, Wq, K_ext, V_ext, Wo):
    x2 = x[0].astype(jnp.bfloat16)
    k2 = K_ext[0].reshape(S, D).astype(jnp.bfloat16)
    v2 = V_ext[0].reshape(S, D).astype(jnp.bfloat16)
    wq = Wq.astype(jnp.bfloat16)
    wo = Wo.astype(jnp.bfloat16)

    def body(x_ref, wq_ref, k_ref, v_ref, wo_ref, out_ref,
             kvf_ref, q_ref, bias_ref, acc_ref, l_ref, send_sems, recv_sems):
        my = lax.axis_index("i")
        left = lax.rem(my + (N_DEV - 1), N_DEV)
        right = lax.rem(my + 1, N_DEV)

        barrier = pltpu.get_barrier_semaphore()
        for nbr in (left, right):
            pl.semaphore_signal(barrier, inc=1, device_id=(nbr,),
                                device_id_type=pl.DeviceIdType.MESH)
        pl.semaphore_wait(barrier, 2)

        kvf_ref[0:S, 0:D] = k_ref[...]
        kvf_ref[0:S, D:2 * D] = v_ref[...]

        def make_hop(h):
            return pltpu.make_async_remote_copy(
                src_ref=kvf_ref.at[pl.ds(h * S, S), :],
                dst_ref=kvf_ref.at[pl.ds((h + 1) * S, S), :],
                send_sem=send_sems.at[h],
                recv_sem=recv_sems.at[h],
                device_id=(right,),
                device_id_type=pl.DeviceIdType.MESH,
            )

        hop0 = make_hop(0)
        hop0.start()

        q_ref[...] = (jnp.dot(
            x_ref[...], wq_ref[...], preferred_element_type=jnp.float32
        ) * SCALE).astype(jnp.bfloat16)

        def compute_chunk(d):
            o = lax.rem(my - d + N_DEV, N_DEV)
            qb = lax.broadcasted_iota(jnp.int32, (S, 1), 0) // BLK \
                + my * (S // BLK)
            kb = lax.broadcasted_iota(jnp.int32, (1, S), 1) // BLK \
                + o * (S // BLK)
            s3 = lax.rem(qb, 3) + lax.rem(kb, 3)
            mask = (qb == kb) | (kb == 0) | (s3 == 0) | (s3 == 3)
            bias_ref[...] = jnp.where(mask, 0.0, -1e9)

            def head_step(hd, _):
                csl = pl.ds(hd * DH, DH)
                qh = q_ref[:, csl]
                kh = kvf_ref[pl.ds(d * S, S), csl]
                sd = lax.dot_general(
                    qh, kh, (((1,), (1,)), ((), ())),
                    preferred_element_type=jnp.float32)
                w = jnp.exp(sd + bias_ref[...])
                lsum = jnp.broadcast_to(
                    jnp.sum(w, axis=1, keepdims=True), (S, DH))
                vh = kvf_ref[pl.ds(d * S, S), pl.ds(D + hd * DH, DH)]
                pv = jnp.dot(w.astype(jnp.bfloat16), vh,
                             preferred_element_type=jnp.float32)
                if d == 0:
                    acc_ref[:, csl] = pv
                    l_ref[:, csl] = lsum
                else:
                    acc_ref[:, csl] = acc_ref[:, csl] + pv
                    l_ref[:, csl] = l_ref[:, csl] + lsum
                return 0

            lax.fori_loop(0, HQ, head_step, 0)

        compute_chunk(0)
        hop0.wait()
        hop1 = make_hop(1)
        hop1.start()
        compute_chunk(1)
        hop1.wait()
        hop2 = make_hop(2)
        hop2.start()
        compute_chunk(2)
        hop2.wait()
        compute_chunk(3)

        def norm_step(hd, _):
            csl = pl.ds(hd * DH, DH)
            q_ref[:, csl] = (acc_ref[:, csl] / l_ref[:, csl]
                             ).astype(jnp.bfloat16)
            return 0

        lax.fori_loop(0, HQ, norm_step, 0)
        out_ref[...] = jnp.dot(
            q_ref[...], wo_ref[...], preferred_element_type=jnp.float32)

    out = pl.pallas_call(
        body,
        out_shape=jax.ShapeDtypeStruct((S, D), jnp.float32),
        in_specs=[pl.BlockSpec(memory_space=pltpu.VMEM)] * 5,
        out_specs=pl.BlockSpec(memory_space=pltpu.VMEM),
        scratch_shapes=[
            pltpu.VMEM((N_DEV * S, 2 * D), jnp.bfloat16),
            pltpu.VMEM((S, D), jnp.bfloat16),
            pltpu.VMEM((S, S), jnp.float32),
            pltpu.VMEM((S, D), jnp.float32),
            pltpu.VMEM((S, D), jnp.float32),
            pltpu.SemaphoreType.DMA((N_DEV - 1,)),
            pltpu.SemaphoreType.DMA((N_DEV - 1,)),
        ],
        compiler_params=pltpu.CompilerParams(
            collective_id=0, vmem_limit_bytes=100 * 1024 * 1024
        ),
    )(x2, wq, k2, v2, wo)
    return out[None]
